# baseline (device time: 119179 ns/iter reference)
import os

import jax
import jax.numpy as jnp
from jax import lax
from jax.experimental import pallas as pl
from jax.experimental.pallas import tpu as pltpu

_SKIP_COMM = bool(os.environ.get("SCB_SKIP_COMM"))

N_DEV = 4
_HB = 1024
_N_LAYERS = 3
_NSLOT = 2
_LOOKAHEAD = 1


def _mod(a, n):
    return lax.rem(a + n, n)


def kernel(x, Win0, Wout0, Win1, Wout1, Win2, Wout2):
    m_per, d = x.shape
    _, h_per = Win0.shape
    M = N_DEV * m_per
    nblk = h_per // _HB
    blocks = [(lyr, kb) for lyr in range(_N_LAYERS) for kb in range(nblk)]

    def body(x_ref, w0i_ref, w0o_ref, w1i_ref, w1o_ref, w2i_ref, w2o_ref,
             out_ref,
             xg_ref, p_ref, xbuf_ref, rs_out, rs_in, win_st, wout_st,
             ag_s, ag_r, rs_s, rs_r, wi_sem, wo_sem):
        i = lax.axis_index("i")
        L_dev = _mod(i - 1, N_DEV)
        R_dev = _mod(i + 1, N_DEV)
        D_dev = _mod(i + 2, N_DEV)
        win_refs = [w0i_ref, w1i_ref, w2i_ref]
        wout_refs = [w0o_ref, w1o_ref, w2o_ref]

        def chunk(c):
            return pl.ds(_mod(c, N_DEV) * m_per, m_per)

        def pair(idx):
            lyr, kb = blocks[idx]
            slot = idx % _NSLOT
            c1 = pltpu.make_async_copy(
                win_refs[lyr].at[:, pl.ds(kb * _HB, _HB)],
                win_st.at[slot], wi_sem.at[slot])
            c2 = pltpu.make_async_copy(
                wout_refs[lyr].at[pl.ds(kb * _HB, _HB), :],
                wout_st.at[slot], wo_sem.at[slot])
            return c1, c2

        for idx0 in range(_LOOKAHEAD):
            c1, c2 = pair(idx0)
            c1.start()
            c2.start()

        bar = pltpu.get_barrier_semaphore()
        for nbr in (L_dev, R_dev, D_dev):
            pl.semaphore_signal(bar, inc=1, device_id=(nbr,),
                                device_id_type=pl.DeviceIdType.MESH)
        pl.semaphore_wait(bar, 3)

        for lyr in range(_N_LAYERS):
            xin = x_ref[...] if lyr == 0 else xbuf_ref[...]
            xg_ref[chunk(i), :] = xin.astype(jnp.bfloat16)

            def do_allgather():
                ds_ = []
                for k, dev in enumerate((R_dev, L_dev, D_dev)):
                    dd = pltpu.make_async_remote_copy(
                        src_ref=xg_ref.at[chunk(i), :],
                        dst_ref=xg_ref.at[chunk(i), :],
                        send_sem=ag_s.at[k], recv_sem=ag_r.at[k],
                        device_id=(dev,),
                        device_id_type=pl.DeviceIdType.MESH)
                    dd.start()
                    ds_.append(dd)
                for dd in ds_:
                    dd.wait_recv()
                for dd in ds_:
                    dd.wait_send()

            if not _SKIP_COMM:
                do_allgather()

            xg = xg_ref[...]
            acc = jnp.zeros((M, d), jnp.float32)
            for kb in range(nblk):
                idx = lyr * nblk + kb
                c1, c2 = pair(idx)
                c1.wait()
                c2.wait()
                if idx + _LOOKAHEAD < len(blocks):
                    n1, n2 = pair(idx + _LOOKAHEAD)
                    n1.start()
                    n2.start()
                slot = idx % _NSLOT
                w1 = win_st[slot, :, :].astype(jnp.bfloat16)
                hb = jnp.dot(xg, w1, preferred_element_type=jnp.float32)
                hb = jnp.maximum(hb, 0.0).astype(jnp.bfloat16)
                w2 = wout_st[slot, :, :].astype(jnp.bfloat16)
                acc = acc + jnp.dot(hb, w2,
                                    preferred_element_type=jnp.float32)
            p_ref[...] = acc

            def do_reduce_scatter():
                rs_out[0, :, :] = p_ref[chunk(i + 1), :].astype(jnp.bfloat16)
                rs_out[1, :, :] = p_ref[chunk(i - 1), :].astype(jnp.bfloat16)
                rs_out[2, :, :] = p_ref[chunk(i + 2), :].astype(jnp.bfloat16)
                es = []
                for k, dev in enumerate((R_dev, L_dev, D_dev)):
                    ee = pltpu.make_async_remote_copy(
                        src_ref=rs_out.at[k], dst_ref=rs_in.at[k],
                        send_sem=rs_s.at[k], recv_sem=rs_r.at[k],
                        device_id=(dev,),
                        device_id_type=pl.DeviceIdType.MESH)
                    ee.start()
                    es.append(ee)
                for ee in es:
                    ee.wait_recv()
                res = (p_ref[chunk(i), :]
                       + rs_in[0, :, :].astype(jnp.float32)
                       + rs_in[1, :, :].astype(jnp.float32)
                       + rs_in[2, :, :].astype(jnp.float32))
                for ee in es:
                    ee.wait_send()
                return res

            if _SKIP_COMM:
                res = p_ref[chunk(i), :]
            else:
                res = do_reduce_scatter()
            if lyr == _N_LAYERS - 1:
                out_ref[...] = res
            else:
                xbuf_ref[...] = res

    return pl.pallas_call(
        body,
        out_shape=jax.ShapeDtypeStruct((m_per, d), jnp.float32),
        in_specs=[
            pl.BlockSpec(memory_space=pltpu.VMEM),
            pl.BlockSpec(memory_space=pltpu.HBM),
            pl.BlockSpec(memory_space=pltpu.HBM),
            pl.BlockSpec(memory_space=pltpu.HBM),
            pl.BlockSpec(memory_space=pltpu.HBM),
            pl.BlockSpec(memory_space=pltpu.HBM),
            pl.BlockSpec(memory_space=pltpu.HBM),
        ],
        out_specs=pl.BlockSpec(memory_space=pltpu.VMEM),
        scratch_shapes=[
            pltpu.VMEM((M, d), jnp.bfloat16),
            pltpu.VMEM((M, d), jnp.float32),
            pltpu.VMEM((m_per, d), jnp.float32),
            pltpu.VMEM((3, m_per, d), jnp.bfloat16),
            pltpu.VMEM((3, m_per, d), jnp.bfloat16),
            pltpu.VMEM((_NSLOT, d, _HB), jnp.float32),
            pltpu.VMEM((_NSLOT, _HB, d), jnp.float32),
            pltpu.SemaphoreType.DMA((3,)),
            pltpu.SemaphoreType.DMA((3,)),
            pltpu.SemaphoreType.DMA((3,)),
            pltpu.SemaphoreType.DMA((3,)),
            pltpu.SemaphoreType.DMA((_NSLOT,)),
            pltpu.SemaphoreType.DMA((_NSLOT,)),
        ],
        compiler_params=pltpu.CompilerParams(
            collective_id=0,
            vmem_limit_bytes=60 * 1024 * 1024,
        ),
    )(x, Win0, Wout0, Win1, Wout1, Win2, Wout2)


# device time: 94678 ns/iter; 1.2588x vs baseline; 1.2588x over previous
import os

import jax
import jax.numpy as jnp
from jax import lax
from jax.experimental import pallas as pl
from jax.experimental.pallas import tpu as pltpu

_SKIP_COMM = bool(os.environ.get("SCB_SKIP_COMM"))

N_DEV = 4
_HB = 512
_N_LAYERS = 3
_NSLOT = 6
_LOOKAHEAD = 5


def _mod(a, n):
    return lax.rem(a + n, n)


def kernel(x, Win0, Wout0, Win1, Wout1, Win2, Wout2):
    m_per, d = x.shape
    _, h_per = Win0.shape
    M = N_DEV * m_per
    nblk = h_per // _HB
    blocks = [(lyr, kb) for lyr in range(_N_LAYERS) for kb in range(nblk)]

    def body(x_ref, w0i_ref, w0o_ref, w1i_ref, w1o_ref, w2i_ref, w2o_ref,
             out_ref,
             xg_ref, p_ref, xbuf_ref, rs_out, rs_in, win_st, wout_st,
             ag_s, ag_r, rs_s, rs_r, wi_sem, wo_sem):
        i = lax.axis_index("i")
        L_dev = _mod(i - 1, N_DEV)
        R_dev = _mod(i + 1, N_DEV)
        D_dev = _mod(i + 2, N_DEV)
        win_refs = [w0i_ref, w1i_ref, w2i_ref]
        wout_refs = [w0o_ref, w1o_ref, w2o_ref]

        def chunk(c):
            return pl.ds(_mod(c, N_DEV) * m_per, m_per)

        def pair(idx):
            lyr, kb = blocks[idx]
            slot = idx % _NSLOT
            c1 = pltpu.make_async_copy(
                win_refs[lyr].at[:, pl.ds(kb * _HB, _HB)],
                win_st.at[slot], wi_sem.at[slot])
            c2 = pltpu.make_async_copy(
                wout_refs[lyr].at[pl.ds(kb * _HB, _HB), :],
                wout_st.at[slot], wo_sem.at[slot])
            return c1, c2

        for idx0 in range(_LOOKAHEAD):
            c1, c2 = pair(idx0)
            c1.start()
            c2.start()

        bar = pltpu.get_barrier_semaphore()
        for nbr in (L_dev, R_dev, D_dev):
            pl.semaphore_signal(bar, inc=1, device_id=(nbr,),
                                device_id_type=pl.DeviceIdType.MESH)
        pl.semaphore_wait(bar, 3)

        for lyr in range(_N_LAYERS):
            xin = x_ref[...] if lyr == 0 else xbuf_ref[...]
            xg_ref[chunk(i), :] = xin.astype(jnp.bfloat16)

            def do_allgather():
                ds_ = []
                for k, dev in enumerate((R_dev, L_dev, D_dev)):
                    dd = pltpu.make_async_remote_copy(
                        src_ref=xg_ref.at[chunk(i), :],
                        dst_ref=xg_ref.at[chunk(i), :],
                        send_sem=ag_s.at[k], recv_sem=ag_r.at[k],
                        device_id=(dev,),
                        device_id_type=pl.DeviceIdType.MESH)
                    dd.start()
                    ds_.append(dd)
                for dd in ds_:
                    dd.wait_recv()
                for dd in ds_:
                    dd.wait_send()

            if not _SKIP_COMM:
                do_allgather()

            xg = xg_ref[...]
            acc = jnp.zeros((M, d), jnp.float32)
            for kb in range(nblk):
                idx = lyr * nblk + kb
                c1, c2 = pair(idx)
                c1.wait()
                c2.wait()
                if idx + _LOOKAHEAD < len(blocks):
                    n1, n2 = pair(idx + _LOOKAHEAD)
                    n1.start()
                    n2.start()
                slot = idx % _NSLOT
                w1 = win_st[slot, :, :].astype(jnp.bfloat16)
                hb = jnp.dot(xg, w1, preferred_element_type=jnp.float32)
                hb = jnp.maximum(hb, 0.0).astype(jnp.bfloat16)
                w2 = wout_st[slot, :, :].astype(jnp.bfloat16)
                acc = acc + jnp.dot(hb, w2,
                                    preferred_element_type=jnp.float32)
            p_ref[...] = acc

            def do_reduce_scatter():
                rs_out[0, :, :] = p_ref[chunk(i + 1), :].astype(jnp.bfloat16)
                rs_out[1, :, :] = p_ref[chunk(i - 1), :].astype(jnp.bfloat16)
                rs_out[2, :, :] = p_ref[chunk(i + 2), :].astype(jnp.bfloat16)
                es = []
                for k, dev in enumerate((R_dev, L_dev, D_dev)):
                    ee = pltpu.make_async_remote_copy(
                        src_ref=rs_out.at[k], dst_ref=rs_in.at[k],
                        send_sem=rs_s.at[k], recv_sem=rs_r.at[k],
                        device_id=(dev,),
                        device_id_type=pl.DeviceIdType.MESH)
                    ee.start()
                    es.append(ee)
                for ee in es:
                    ee.wait_recv()
                res = (p_ref[chunk(i), :]
                       + rs_in[0, :, :].astype(jnp.float32)
                       + rs_in[1, :, :].astype(jnp.float32)
                       + rs_in[2, :, :].astype(jnp.float32))
                for ee in es:
                    ee.wait_send()
                return res

            if _SKIP_COMM:
                res = p_ref[chunk(i), :]
            else:
                res = do_reduce_scatter()
            if lyr == _N_LAYERS - 1:
                out_ref[...] = res
            else:
                xbuf_ref[...] = res

    return pl.pallas_call(
        body,
        out_shape=jax.ShapeDtypeStruct((m_per, d), jnp.float32),
        in_specs=[
            pl.BlockSpec(memory_space=pltpu.VMEM),
            pl.BlockSpec(memory_space=pltpu.HBM),
            pl.BlockSpec(memory_space=pltpu.HBM),
            pl.BlockSpec(memory_space=pltpu.HBM),
            pl.BlockSpec(memory_space=pltpu.HBM),
            pl.BlockSpec(memory_space=pltpu.HBM),
            pl.BlockSpec(memory_space=pltpu.HBM),
        ],
        out_specs=pl.BlockSpec(memory_space=pltpu.VMEM),
        scratch_shapes=[
            pltpu.VMEM((M, d), jnp.bfloat16),
            pltpu.VMEM((M, d), jnp.float32),
            pltpu.VMEM((m_per, d), jnp.float32),
            pltpu.VMEM((3, m_per, d), jnp.bfloat16),
            pltpu.VMEM((3, m_per, d), jnp.bfloat16),
            pltpu.VMEM((_NSLOT, d, _HB), jnp.float32),
            pltpu.VMEM((_NSLOT, _HB, d), jnp.float32),
            pltpu.SemaphoreType.DMA((3,)),
            pltpu.SemaphoreType.DMA((3,)),
            pltpu.SemaphoreType.DMA((3,)),
            pltpu.SemaphoreType.DMA((3,)),
            pltpu.SemaphoreType.DMA((_NSLOT,)),
            pltpu.SemaphoreType.DMA((_NSLOT,)),
        ],
        compiler_params=pltpu.CompilerParams(
            collective_id=0,
            vmem_limit_bytes=62 * 1024 * 1024,
        ),
    )(x, Win0, Wout0, Win1, Wout1, Win2, Wout2)
